# baseline (speedup 1.0000x reference)
import functools

import jax
import jax.numpy as jnp
from jax import lax
from jax.experimental import pallas as pl
from jax.experimental.pallas import tpu as pltpu
from jax.experimental.pallas import tpu_sc as plsc

_N = 8192
_D = 256
_BR = 512
_NR = _N // _BR
_BETA = 0.25


def _vq_body(zf_ref, wt2_ref, zn_ref, wn_ref,
             idx_ref, enc_ref, loss_ref, cnt_ref,
             bi_ref, bip_ref):
    i = pl.program_id(0)
    idsi = lax.broadcasted_iota(jnp.int32, (1, _N), 1)
    lane = lax.broadcasted_iota(jnp.int32, (1, 128), 1).astype(jnp.float32)

    bip_ref[...] = bi_ref[...]

    s2 = lax.dot_general(zf_ref[...], wt2_ref[...],
                         (((1,), (0,)), ((), ())),
                         preferred_element_type=jnp.float32)
    zn = zn_ref[...]
    m = None
    av = None
    for k in range(_N // 128):
        dk = ((zn + wn_ref[0:1, pl.ds(k * 128, 128)])
              + s2[:, k * 128:(k + 1) * 128])
        colv = lane + jnp.float32(k * 128)
        if m is None:
            m = dk
            av = jnp.broadcast_to(colv, dk.shape)
        else:
            upd = dk < m
            m = jnp.where(upd, dk, m)
            av = jnp.where(upd, colv, av)
    mrow = jnp.min(m, axis=1, keepdims=True)
    a_loc = jnp.min(jnp.where(m == mrow, av, jnp.float32(3e38)),
                    axis=1, keepdims=True)
    a = a_loc.astype(jnp.int32)
    idx_ref[...] = a
    bi_ref[...] = a
    part = jnp.sum(mrow)

    @pl.when(i == 0)
    def _():
        loss_ref[0, 0] = part

    @pl.when(i > 0)
    def _():
        loss_ref[0, 0] = loss_ref[0, 0] + part

    @pl.when(i > 0)
    def _emit():
        oh = (bip_ref[...] == idsi).astype(jnp.float32)
        enc_ref[...] = oh
        col = jnp.sum(oh, axis=0, keepdims=True)

        @pl.when(i == 1)
        def _():
            cnt_ref[...] = col

        @pl.when(i > 1)
        def _():
            cnt_ref[...] = cnt_ref[...] + col


def _fin_body(bi_ref, cnt_ref, lraw_ref, _enc_in_ref,
              enc_ref, loss_ref, perp_ref):
    idsi = lax.broadcasted_iota(jnp.int32, (1, _N), 1)
    oh = (bi_ref[...] == idsi).astype(jnp.float32)
    enc_ref[...] = oh
    col = jnp.sum(oh, axis=0, keepdims=True)
    cnt = cnt_ref[...] + col
    p = cnt * (1.0 / _N)
    ent = jnp.sum(p * jnp.log(p + 1e-10))
    perp_ref[0, 0] = jnp.exp(-ent)
    loss_ref[0, 0] = lraw_ref[0, 0] * ((1.0 + _BETA) / (_N * _D))


def _tr_body(zq_ref, out_ref):
    out_ref[0, :, :] = zq_ref[0, :, :].T


def _make_sc_gather(num_cores, num_subcores):
    nw = num_cores * num_subcores
    bpw = _N // nw
    mesh = plsc.VectorSubcoreMesh(core_axis_name="c", subcore_axis_name="s")

    @functools.partial(
        pl.kernel, mesh=mesh,
        out_type=jax.ShapeDtypeStruct((_N, _D), jnp.float32),
        scratch_types=[
            pltpu.VMEM((bpw,), jnp.int32),
            pltpu.VMEM((bpw, _D), jnp.float32),
            pltpu.SemaphoreType.DMA,
        ],
    )
    def gather(table_hbm, idx_hbm, out_hbm, idx_v, rows_v, sem):
        wid = lax.axis_index("s") * num_cores + lax.axis_index("c")
        base = wid * bpw
        pltpu.sync_copy(idx_hbm.at[pl.ds(base, bpw)], idx_v)
        pltpu.async_copy(table_hbm.at[idx_v], rows_v, sem).wait()
        pltpu.sync_copy(rows_v, out_hbm.at[pl.ds(base, bpw)])

    return gather


_vq_call = pl.pallas_call(
    _vq_body,
    grid=(_NR,),
    in_specs=[
        pl.BlockSpec((_BR, _D), lambda i: (i, 0)),
        pl.BlockSpec((_D, _N), lambda i: (0, 0)),
        pl.BlockSpec((_BR, 1), lambda i: (i, 0)),
        pl.BlockSpec((1, _N), lambda i: (0, 0)),
    ],
    out_specs=[
        pl.BlockSpec((_BR, 1), lambda i: (i, 0)),
        pl.BlockSpec((_BR, _N), lambda i: (jnp.maximum(i - 1, 0), 0)),
        pl.BlockSpec(memory_space=pltpu.SMEM),
        pl.BlockSpec((1, _N), lambda i: (0, 0)),
    ],
    out_shape=[
        jax.ShapeDtypeStruct((_N, 1), jnp.int32),
        jax.ShapeDtypeStruct((_N, _N), jnp.float32),
        jax.ShapeDtypeStruct((1, 1), jnp.float32),
        jax.ShapeDtypeStruct((1, _N), jnp.float32),
    ],
    scratch_shapes=[
        pltpu.VMEM((_BR, 1), jnp.int32),
        pltpu.VMEM((_BR, 1), jnp.int32),
    ],
    compiler_params=pltpu.CompilerParams(
        dimension_semantics=("arbitrary",)),
)

_fin_call = pl.pallas_call(
    _fin_body,
    grid=(1,),
    in_specs=[
        pl.BlockSpec((_BR, 1), lambda j: (_NR - 1, 0)),
        pl.BlockSpec((1, _N), lambda j: (0, 0)),
        pl.BlockSpec(memory_space=pltpu.SMEM),
        pl.BlockSpec((8, 128), lambda j: (0, 0)),
    ],
    out_specs=[
        pl.BlockSpec((_BR, _N), lambda j: (_NR - 1, 0)),
        pl.BlockSpec(memory_space=pltpu.SMEM),
        pl.BlockSpec(memory_space=pltpu.SMEM),
    ],
    out_shape=[
        jax.ShapeDtypeStruct((_N, _N), jnp.float32),
        jax.ShapeDtypeStruct((1, 1), jnp.float32),
        jax.ShapeDtypeStruct((1, 1), jnp.float32),
    ],
    input_output_aliases={3: 0},
)

_tr_call = pl.pallas_call(
    _tr_body,
    grid=(8,),
    in_specs=[pl.BlockSpec((1, _N // 8, _D), lambda j: (j, 0, 0))],
    out_specs=pl.BlockSpec((1, _D, _N // 8), lambda j: (j, 0, 0)),
    out_shape=jax.ShapeDtypeStruct((8, _D, _N // 8), jnp.float32),
)


def kernel(z, W):
    b, cdim, h, w = z.shape
    zt = jnp.transpose(z, (0, 2, 3, 1))
    zf = zt.reshape(-1, _D)
    znorm = jnp.sum(zf ** 2, axis=1, keepdims=True)
    wnorm = jnp.sum(W ** 2, axis=1).reshape(1, _N)
    wt2 = (-2.0 * W).T
    idx2, enc0, lraw, cnt = _vq_call(zf, wt2, znorm, wnorm)
    idx = idx2.reshape(_N)

    info = plsc.get_sparse_core_info()
    zq = _make_sc_gather(info.num_cores, info.num_subcores)(W, idx)

    enc, loss, perp = _fin_call(idx2, cnt, lraw, enc0)
    zqt = _tr_call(zq.reshape(b, h * w, cdim))
    z_q_out = zqt.reshape(b, cdim, h, w)
    return (z_q_out, loss[0, 0], perp[0, 0], enc, idx)
